# baseline (speedup 1.0000x reference)
import functools

import jax
import jax.numpy as jnp
from jax import lax
from jax.experimental import pallas as pl
from jax.experimental.pallas import tpu as pltpu
from jax.experimental.pallas import tpu_sc as plsc

_EPS = 1e-12


def _sc_gather(ids2d, word_table):
    n_rows, CH = ids2d.shape
    H = word_table.shape[1]
    info = plsc.get_sparse_core_info()
    NC, NS = info.num_cores, info.num_subcores
    NW = NC * NS
    n_ch = n_rows // NW
    NB = next(b for b in (5, 4, 3, 2, 1) if n_ch % b == 0)
    N = n_rows * CH

    mesh = plsc.VectorSubcoreMesh(core_axis_name="c", subcore_axis_name="s")
    ids3d = ids2d.reshape(NW, n_ch, CH)

    @functools.partial(
        pl.kernel,
        mesh=mesh,
        out_type=jax.ShapeDtypeStruct((N, H), jnp.float32),
        scratch_types=[
            pltpu.VMEM((n_ch, CH), jnp.int32),
            pltpu.VMEM((NB, CH, H), jnp.float32),
        ]
        + [pltpu.SemaphoreType.DMA] * (2 * NB),
    )
    def k(ids_hbm, table_hbm, out_hbm, idx_v, rows_v, *sems):
        wid = lax.axis_index("s") * NC + lax.axis_index("c")
        rbase = wid * n_ch
        gsem = sems[:NB]
        wsem = sems[NB:]

        pltpu.sync_copy(ids_hbm.at[wid], idx_v)
        for b in range(NB):
            pltpu.async_copy(table_hbm.at[idx_v.at[b]], rows_v.at[b], gsem[b])

        def body(i, carry):
            for b in range(NB):
                ch = i * NB + b
                off = (rbase + ch) * CH
                pltpu.make_async_copy(
                    table_hbm.at[idx_v.at[ch]], rows_v.at[b], gsem[b]
                ).wait()
                pltpu.async_copy(
                    rows_v.at[b], out_hbm.at[pl.ds(off, CH)], wsem[b]
                )

                @pl.when(i < n_ch // NB - 1)
                def _prefetch():
                    pltpu.make_async_copy(
                        rows_v.at[b], out_hbm.at[pl.ds(off, CH)], wsem[b]
                    ).wait()
                    pltpu.async_copy(
                        table_hbm.at[idx_v.at[ch + NB]], rows_v.at[b], gsem[b]
                    )

            return carry

        lax.fori_loop(0, n_ch // NB, body, 0)
        for b in range(NB):
            off = (rbase + n_ch - NB + b) * CH
            pltpu.make_async_copy(
                rows_v.at[b], out_hbm.at[pl.ds(off, CH)], wsem[b]
            ).wait()

    return k(ids3d, word_table)


def _tc_add_layernorm(gathered, pre_tab, pos_tab, gamma, beta):
    B, L, H = gathered.shape
    RB = 32
    grid = (B // RB,)

    def body(g_ref, pa_ref, pb_ref, gm_ref, bt_ref, o_ref):
        x = g_ref[...] + (pa_ref[...] + pb_ref[...])[None, :, :]
        u = jnp.mean(x, axis=-1, keepdims=True)
        s2 = jnp.mean((x - u) ** 2, axis=-1, keepdims=True)
        xn = (x - u) * lax.rsqrt(s2 + _EPS)
        o_ref[...] = xn * gm_ref[0][None, None, :] + bt_ref[0][None, None, :]

    return pl.pallas_call(
        body,
        grid=grid,
        in_specs=[
            pl.BlockSpec((RB, L, H), lambda i: (i, 0, 0)),
            pl.BlockSpec((L, H), lambda i: (0, 0)),
            pl.BlockSpec((L, H), lambda i: (0, 0)),
            pl.BlockSpec((1, H), lambda i: (0, 0)),
            pl.BlockSpec((1, H), lambda i: (0, 0)),
        ],
        out_specs=pl.BlockSpec((RB, L, H), lambda i: (i, 0, 0)),
        out_shape=jax.ShapeDtypeStruct((B, L, H), jnp.float32),
    )(gathered, pre_tab, pos_tab, gamma, beta)


def kernel(input_ids, word_table, pretrained_table, pos_table, gamma, beta):
    B, L = input_ids.shape
    H = word_table.shape[1]
    CH = 128
    ids2d = input_ids.reshape(-1).astype(jnp.int32).reshape(-1, CH)
    pre = pretrained_table[:L]
    pos = pos_table[:L]
    gm = gamma.reshape(1, H)
    bt = beta.reshape(1, H)

    gathered = _sc_gather(ids2d, word_table)
    return _tc_add_layernorm(gathered.reshape(B, L, H), pre, pos, gm, bt)


# speedup vs baseline: 1.0454x; 1.0454x over previous
import functools

import jax
import jax.numpy as jnp
from jax import lax
from jax.experimental import pallas as pl
from jax.experimental.pallas import tpu as pltpu
from jax.experimental.pallas import tpu_sc as plsc

_EPS = 1e-12


def _sc_gather(ids2d, word_table):
    n_rows, CH = ids2d.shape
    H = word_table.shape[1]
    info = plsc.get_sparse_core_info()
    NC, NS = info.num_cores, info.num_subcores
    NW = NC * NS
    n_ch = n_rows // NW
    NB = next(b for b in (5, 4, 3, 2, 1) if n_ch % b == 0)
    N = n_rows * CH

    mesh = plsc.VectorSubcoreMesh(core_axis_name="c", subcore_axis_name="s")
    ids3d = ids2d.reshape(NW, n_ch, CH)

    @functools.partial(
        pl.kernel,
        mesh=mesh,
        out_type=jax.ShapeDtypeStruct((N, H), jnp.float32),
        scratch_types=[
            pltpu.VMEM((n_ch, CH), jnp.int32),
            pltpu.VMEM((NB, CH, H), jnp.float32),
        ]
        + [pltpu.SemaphoreType.DMA] * (2 * NB),
    )
    def k(ids_hbm, table_hbm, out_hbm, idx_v, rows_v, *sems):
        wid = lax.axis_index("s") * NC + lax.axis_index("c")
        rbase = wid * n_ch
        gsem = sems[:NB]
        wsem = sems[NB:]

        pltpu.sync_copy(ids_hbm.at[wid], idx_v)
        for b in range(NB):
            pltpu.async_copy(table_hbm.at[idx_v.at[b]], rows_v.at[b], gsem[b])

        def body(i, carry):
            for b in range(NB):
                ch = i * NB + b
                off = (rbase + ch) * CH
                pltpu.make_async_copy(
                    table_hbm.at[idx_v.at[ch]], rows_v.at[b], gsem[b]
                ).wait()
                pltpu.async_copy(
                    rows_v.at[b], out_hbm.at[pl.ds(off, CH)], wsem[b]
                )

                @pl.when(i < n_ch // NB - 1)
                def _prefetch():
                    pltpu.make_async_copy(
                        rows_v.at[b], out_hbm.at[pl.ds(off, CH)], wsem[b]
                    ).wait()
                    pltpu.async_copy(
                        table_hbm.at[idx_v.at[ch + NB]], rows_v.at[b], gsem[b]
                    )

            return carry

        lax.fori_loop(0, n_ch // NB, body, 0)
        for b in range(NB):
            off = (rbase + n_ch - NB + b) * CH
            pltpu.make_async_copy(
                rows_v.at[b], out_hbm.at[pl.ds(off, CH)], wsem[b]
            ).wait()

    return k(ids3d, word_table)


def _tc_add_layernorm(gathered, pre_tab, pos_tab, gamma, beta):
    B, L, H = gathered.shape
    RB = 64
    grid = (B // RB,)

    def body(g_ref, pa_ref, pb_ref, gm_ref, bt_ref, o_ref):
        x = g_ref[...] + (pa_ref[...] + pb_ref[...])[None, :, :]
        u = jnp.mean(x, axis=-1, keepdims=True)
        s2 = jnp.mean((x - u) ** 2, axis=-1, keepdims=True)
        xn = (x - u) * lax.rsqrt(s2 + _EPS)
        o_ref[...] = xn * gm_ref[0][None, None, :] + bt_ref[0][None, None, :]

    return pl.pallas_call(
        body,
        grid=grid,
        in_specs=[
            pl.BlockSpec((RB, L, H), lambda i: (i, 0, 0)),
            pl.BlockSpec((L, H), lambda i: (0, 0)),
            pl.BlockSpec((L, H), lambda i: (0, 0)),
            pl.BlockSpec((1, H), lambda i: (0, 0)),
            pl.BlockSpec((1, H), lambda i: (0, 0)),
        ],
        out_specs=pl.BlockSpec((RB, L, H), lambda i: (i, 0, 0)),
        out_shape=jax.ShapeDtypeStruct((B, L, H), jnp.float32),
        input_output_aliases={0: 0},
    )(gathered, pre_tab, pos_tab, gamma, beta)


def kernel(input_ids, word_table, pretrained_table, pos_table, gamma, beta):
    B, L = input_ids.shape
    H = word_table.shape[1]
    CH = 128
    ids2d = input_ids.reshape(-1).astype(jnp.int32).reshape(-1, CH)
    pre = pretrained_table[:L]
    pos = pos_table[:L]
    gm = gamma.reshape(1, H)
    bt = beta.reshape(1, H)

    gathered = _sc_gather(ids2d, word_table)
    return _tc_add_layernorm(gathered.reshape(B, L, H), pre, pos, gm, bt)


# speedup vs baseline: 1.0501x; 1.0045x over previous
import functools

import jax
import jax.numpy as jnp
from jax import lax
from jax.experimental import pallas as pl
from jax.experimental.pallas import tpu as pltpu
from jax.experimental.pallas import tpu_sc as plsc

_EPS = 1e-12


def _sc_gather(ids2d, word_table):
    n_rows, CH = ids2d.shape
    H = word_table.shape[1]
    info = plsc.get_sparse_core_info()
    NC, NS = info.num_cores, info.num_subcores
    NW = NC * NS
    n_ch = n_rows // NW
    NB = next(b for b in (5, 4, 3, 2, 1) if n_ch % b == 0)
    N = n_rows * CH

    mesh = plsc.VectorSubcoreMesh(core_axis_name="c", subcore_axis_name="s")
    ids3d = ids2d.reshape(NW, n_ch, CH)

    @functools.partial(
        pl.kernel,
        mesh=mesh,
        out_type=jax.ShapeDtypeStruct((N, H), jnp.float32),
        scratch_types=[
            pltpu.VMEM((n_ch, CH), jnp.int32),
            pltpu.VMEM((NB, CH, H), jnp.float32),
        ]
        + [pltpu.SemaphoreType.DMA] * (2 * NB),
    )
    def k(ids_hbm, table_hbm, out_hbm, idx_v, rows_v, *sems):
        wid = lax.axis_index("s") * NC + lax.axis_index("c")
        rbase = wid * n_ch
        gsem = sems[:NB]
        wsem = sems[NB:]

        pltpu.sync_copy(ids_hbm.at[wid], idx_v)
        for b in range(NB):
            pltpu.async_copy(table_hbm.at[idx_v.at[b]], rows_v.at[b], gsem[b])

        def body(i, carry):
            for b in range(NB):
                ch = i * NB + b
                off = (rbase + ch) * CH
                pltpu.make_async_copy(
                    table_hbm.at[idx_v.at[ch]], rows_v.at[b], gsem[b]
                ).wait()
                pltpu.async_copy(
                    rows_v.at[b], out_hbm.at[pl.ds(off, CH)], wsem[b]
                )

                @pl.when(i < n_ch // NB - 1)
                def _prefetch():
                    pltpu.make_async_copy(
                        rows_v.at[b], out_hbm.at[pl.ds(off, CH)], wsem[b]
                    ).wait()
                    pltpu.async_copy(
                        table_hbm.at[idx_v.at[ch + NB]], rows_v.at[b], gsem[b]
                    )

            return carry

        lax.fori_loop(0, n_ch // NB, body, 0)
        for b in range(NB):
            off = (rbase + n_ch - NB + b) * CH
            pltpu.make_async_copy(
                rows_v.at[b], out_hbm.at[pl.ds(off, CH)], wsem[b]
            ).wait()

    return k(ids3d, word_table)


def _tc_add_layernorm(gathered, pre_tab, pos_tab, gamma, beta):
    B, L, H = gathered.shape
    RB = 64
    grid = (B // RB,)

    def body(g_ref, pa_ref, pb_ref, gm_ref, bt_ref, o_ref):
        x = g_ref[...] + (pa_ref[...] + pb_ref[...])[None, :, :]
        u = jnp.mean(x, axis=-1, keepdims=True)
        s2 = jnp.mean((x - u) ** 2, axis=-1, keepdims=True)
        xn = (x - u) * lax.rsqrt(s2 + _EPS)
        o_ref[...] = xn * gm_ref[0][None, None, :] + bt_ref[0][None, None, :]

    return pl.pallas_call(
        body,
        grid=grid,
        in_specs=[
            pl.BlockSpec((RB, L, H), lambda i: (i, 0, 0)),
            pl.BlockSpec((L, H), lambda i: (0, 0)),
            pl.BlockSpec((L, H), lambda i: (0, 0)),
            pl.BlockSpec((1, H), lambda i: (0, 0)),
            pl.BlockSpec((1, H), lambda i: (0, 0)),
        ],
        out_specs=pl.BlockSpec((RB, L, H), lambda i: (i, 0, 0)),
        out_shape=jax.ShapeDtypeStruct((B, L, H), jnp.float32),
        compiler_params=pltpu.CompilerParams(
            dimension_semantics=("parallel",)
        ),
    )(gathered, pre_tab, pos_tab, gamma, beta)


def kernel(input_ids, word_table, pretrained_table, pos_table, gamma, beta):
    B, L = input_ids.shape
    H = word_table.shape[1]
    CH = 128
    ids2d = input_ids.reshape(-1).astype(jnp.int32).reshape(-1, CH)
    pre = pretrained_table[:L]
    pos = pos_table[:L]
    gm = gamma.reshape(1, H)
    bt = beta.reshape(1, H)

    gathered = _sc_gather(ids2d, word_table)
    return _tc_add_layernorm(gathered.reshape(B, L, H), pre, pos, gm, bt)


# speedup vs baseline: 1.0518x; 1.0016x over previous
import functools

import jax
import jax.numpy as jnp
from jax import lax
from jax.experimental import pallas as pl
from jax.experimental.pallas import tpu as pltpu
from jax.experimental.pallas import tpu_sc as plsc

_EPS = 1e-12


def _sc_gather(ids2d, word_table):
    n_rows, CH = ids2d.shape
    H = word_table.shape[1]
    info = plsc.get_sparse_core_info()
    NC, NS = info.num_cores, info.num_subcores
    NW = NC * NS
    n_ch = n_rows // NW
    NB = next(b for b in (5, 4, 3, 2, 1) if n_ch % b == 0)
    N = n_rows * CH

    mesh = plsc.VectorSubcoreMesh(core_axis_name="c", subcore_axis_name="s")
    ids3d = ids2d.reshape(NW, n_ch, CH)

    @functools.partial(
        pl.kernel,
        mesh=mesh,
        out_type=jax.ShapeDtypeStruct((N, H), jnp.float32),
        scratch_types=[
            pltpu.VMEM((n_ch, CH), jnp.int32),
            pltpu.VMEM((NB, CH, H), jnp.float32),
        ]
        + [pltpu.SemaphoreType.DMA] * (2 * NB),
    )
    def k(ids_hbm, table_hbm, out_hbm, idx_v, rows_v, *sems):
        wid = lax.axis_index("s") * NC + lax.axis_index("c")
        rbase = wid * n_ch
        gsem = sems[:NB]
        wsem = sems[NB:]

        pltpu.sync_copy(ids_hbm.at[wid], idx_v)
        for b in range(NB):
            pltpu.async_copy(table_hbm.at[idx_v.at[b]], rows_v.at[b], gsem[b])

        def body(i, carry):
            for b in range(NB):
                ch = i * NB + b
                off = (rbase + ch) * CH
                pltpu.make_async_copy(
                    table_hbm.at[idx_v.at[ch]], rows_v.at[b], gsem[b]
                ).wait()
                pltpu.async_copy(
                    rows_v.at[b], out_hbm.at[pl.ds(off, CH)], wsem[b]
                )

                @pl.when(i < n_ch // NB - 1)
                def _prefetch():
                    pltpu.make_async_copy(
                        rows_v.at[b], out_hbm.at[pl.ds(off, CH)], wsem[b]
                    ).wait()
                    pltpu.async_copy(
                        table_hbm.at[idx_v.at[ch + NB]], rows_v.at[b], gsem[b]
                    )

            return carry

        lax.fori_loop(0, n_ch // NB, body, 0)
        for b in range(NB):
            off = (rbase + n_ch - NB + b) * CH
            pltpu.make_async_copy(
                rows_v.at[b], out_hbm.at[pl.ds(off, CH)], wsem[b]
            ).wait()

    return k(ids3d, word_table)


def _tc_add_layernorm(gathered, pre_tab, pos_tab, gamma, beta):
    B, L, H = gathered.shape
    RB = 64
    grid = (B // RB,)

    def body(g_ref, pa_ref, pb_ref, gm_ref, bt_ref, o_ref):
        x = g_ref[...] + (pa_ref[...] + pb_ref[...])[None, :, :]
        u = jnp.mean(x, axis=-1, keepdims=True)
        s2 = jnp.mean((x - u) ** 2, axis=-1, keepdims=True)
        xn = (x - u) * lax.rsqrt(s2 + _EPS)
        o_ref[...] = xn * gm_ref[0][None, None, :] + bt_ref[0][None, None, :]

    return pl.pallas_call(
        body,
        grid=grid,
        in_specs=[
            pl.BlockSpec((RB, L, H), lambda i: (i, 0, 0)),
            pl.BlockSpec((L, H), lambda i: (0, 0)),
            pl.BlockSpec((L, H), lambda i: (0, 0)),
            pl.BlockSpec((1, H), lambda i: (0, 0)),
            pl.BlockSpec((1, H), lambda i: (0, 0)),
        ],
        out_specs=pl.BlockSpec((RB, L, H), lambda i: (i, 0, 0)),
        out_shape=jax.ShapeDtypeStruct((B, L, H), jnp.float32),
    )(gathered, pre_tab, pos_tab, gamma, beta)


def kernel(input_ids, word_table, pretrained_table, pos_table, gamma, beta):
    B, L = input_ids.shape
    H = word_table.shape[1]
    CH = 128
    ids2d = input_ids.reshape(-1).astype(jnp.int32).reshape(-1, CH)
    pre = pretrained_table[:L]
    pos = pos_table[:L]
    gm = gamma.reshape(1, H)
    bt = beta.reshape(1, H)

    gathered = _sc_gather(ids2d, word_table)
    return _tc_add_layernorm(gathered.reshape(B, L, H), pre, pos, gm, bt)


# speedup vs baseline: 1.0528x; 1.0009x over previous
import functools

import jax
import jax.numpy as jnp
from jax import lax
from jax.experimental import pallas as pl
from jax.experimental.pallas import tpu as pltpu
from jax.experimental.pallas import tpu_sc as plsc

_EPS = 1e-12


def _sc_gather(ids2d, word_table):
    n_rows, CH = ids2d.shape
    H = word_table.shape[1]
    info = plsc.get_sparse_core_info()
    NC, NS = info.num_cores, info.num_subcores
    NW = NC * NS
    n_ch = n_rows // NW
    NB = next(b for b in (5, 4, 3, 2, 1) if n_ch % b == 0)
    N = n_rows * CH

    mesh = plsc.VectorSubcoreMesh(core_axis_name="c", subcore_axis_name="s")
    ids3d = ids2d.reshape(NW, n_ch, CH)

    @functools.partial(
        pl.kernel,
        mesh=mesh,
        out_type=jax.ShapeDtypeStruct((N, H), jnp.float32),
        scratch_types=[
            pltpu.VMEM((n_ch, CH), jnp.int32),
            pltpu.VMEM((NB, CH, H), jnp.float32),
        ]
        + [pltpu.SemaphoreType.DMA] * (2 * NB),
    )
    def k(ids_hbm, table_hbm, out_hbm, idx_v, rows_v, *sems):
        wid = lax.axis_index("s") * NC + lax.axis_index("c")
        rbase = wid * n_ch
        gsem = sems[:NB]
        wsem = sems[NB:]

        pltpu.sync_copy(ids_hbm.at[wid], idx_v)
        for b in range(NB):
            pltpu.async_copy(table_hbm.at[idx_v.at[b]], rows_v.at[b], gsem[b])

        def body(i, carry):
            for b in range(NB):
                ch = i * NB + b
                off = (rbase + ch) * CH
                pltpu.make_async_copy(
                    table_hbm.at[idx_v.at[ch]], rows_v.at[b], gsem[b]
                ).wait()
                pltpu.async_copy(
                    rows_v.at[b], out_hbm.at[pl.ds(off, CH)], wsem[b]
                )

                @pl.when(i < n_ch // NB - 1)
                def _prefetch():
                    pltpu.make_async_copy(
                        rows_v.at[b], out_hbm.at[pl.ds(off, CH)], wsem[b]
                    ).wait()
                    pltpu.async_copy(
                        table_hbm.at[idx_v.at[ch + NB]], rows_v.at[b], gsem[b]
                    )

            return carry

        lax.fori_loop(0, n_ch // NB, body, 0)
        for b in range(NB):
            off = (rbase + n_ch - NB + b) * CH
            pltpu.make_async_copy(
                rows_v.at[b], out_hbm.at[pl.ds(off, CH)], wsem[b]
            ).wait()

    return k(ids3d, word_table)


def _tc_add_layernorm(gathered, pre_tab, pos_tab, gamma, beta):
    B, L, H = gathered.shape
    RB = 64
    grid = (B // RB,)

    def body(g_ref, pa_ref, pb_ref, gm_ref, bt_ref, o_ref):
        x = g_ref[...] + (pa_ref[...] + pb_ref[...])[None, :, :]
        u = jnp.mean(x, axis=-1, keepdims=True)
        s2 = jnp.mean((x - u) ** 2, axis=-1, keepdims=True)
        xn = (x - u) * lax.rsqrt(s2 + _EPS)
        o_ref[...] = xn * gm_ref[0][None, None, :] + bt_ref[0][None, None, :]

    return pl.pallas_call(
        body,
        grid=grid,
        in_specs=[
            pl.BlockSpec((RB, L, H), lambda i: (i, 0, 0)),
            pl.BlockSpec((L, H), lambda i: (0, 0)),
            pl.BlockSpec((L, H), lambda i: (0, 0)),
            pl.BlockSpec((1, H), lambda i: (0, 0)),
            pl.BlockSpec((1, H), lambda i: (0, 0)),
        ],
        out_specs=pl.BlockSpec((RB, L, H), lambda i: (i, 0, 0)),
        out_shape=jax.ShapeDtypeStruct((B, L, H), jnp.float32),
    )(gathered, pre_tab, pos_tab, gamma, beta)


def kernel(input_ids, word_table, pretrained_table, pos_table, gamma, beta):
    B, L = input_ids.shape
    H = word_table.shape[1]
    CH = 128
    ids2d = input_ids.reshape(-1).astype(jnp.int32).reshape(-1, CH)
    gm = gamma.reshape(1, H)
    bt = beta.reshape(1, H)

    gathered = _sc_gather(ids2d, word_table)
    return _tc_add_layernorm(
        gathered.reshape(B, L, H), pretrained_table, pos_table, gm, bt
    )
